# baseline (speedup 1.0000x reference)
import functools

import jax
import jax.numpy as jnp
from jax import lax
from jax.experimental import pallas as pl
from jax.experimental.pallas import tpu as pltpu
from jax.experimental.pallas import tpu_sc as plsc

N = 10000
E = 320000
IN_DIM = 128
H = 16
D = 16
HD = H * D


def _matmul2_tc(a, b1, b2, bm):
    M, K = a.shape
    _, K1 = b1.shape
    _, Nc = b2.shape

    def body(a_ref, b1_ref, b2_ref, o_ref):
        t = jnp.dot(a_ref[...].astype(jnp.bfloat16),
                    b1_ref[...].astype(jnp.bfloat16),
                    preferred_element_type=jnp.float32)
        o_ref[...] = jnp.dot(t.astype(jnp.bfloat16),
                             b2_ref[...].astype(jnp.bfloat16),
                             preferred_element_type=jnp.float32)

    return pl.pallas_call(
        body,
        grid=(M // bm,),
        in_specs=[pl.BlockSpec((bm, K), lambda i: (i, 0)),
                  pl.BlockSpec((K, K1), lambda i: (0, 0)),
                  pl.BlockSpec((K1, Nc), lambda i: (0, 0))],
        out_specs=pl.BlockSpec((bm, Nc), lambda i: (i, 0)),
        out_shape=jax.ShapeDtypeStruct((M, Nc), jnp.float32),
    )(a, b1, b2)


def _sc_edge_pass(h2t, evs, src3, dst3):
    info = plsc.get_sparse_core_info()
    NC, NS = info.num_cores, info.num_subcores
    NW = NC * NS
    EPW = E // NW
    CH = 80
    NCH = EPW // CH
    ZROWS = 250

    mesh = plsc.VectorSubcoreMesh(core_axis_name="c", subcore_axis_name="s")

    @functools.partial(
        pl.kernel, mesh=mesh,
        out_type=jax.ShapeDtypeStruct((NC, N, H), jnp.float32),
        compiler_params=pltpu.CompilerParams(use_tc_tiling_on_sc=False),
        scratch_types=[
            pltpu.VMEM((NCH, CH), jnp.int32),
            pltpu.VMEM((NCH, CH), jnp.int32),
            pltpu.VMEM((2, CH, HD), jnp.float32),
            pltpu.VMEM((2, CH, HD), jnp.float32),
            pltpu.VMEM((2, CH, H), jnp.float32),
            pltpu.VMEM((2, CH, H), jnp.float32),
            pltpu.VMEM((ZROWS, H), jnp.float32),
            pltpu.VMEM_SHARED((N, H), jnp.float32),
            pltpu.SemaphoreType.DMA,
            pltpu.SemaphoreType.DMA,
            pltpu.SemaphoreType.DMA,
            pltpu.SemaphoreType.DMA,
        ],
    )
    def k(h2t_hbm, evs_hbm, src_hbm, dst_hbm, out_hbm,
          sidx, didx, srows, drows, evc, vals, zbuf, acc,
          sem0, sem1, sem2, sem3):
        cid = lax.axis_index("c")
        sid = lax.axis_index("s")
        wid = sid * NC + cid

        def zrow(i, carry):
            zbuf[i, :] = jnp.zeros((H,), jnp.float32)
            return carry
        lax.fori_loop(0, ZROWS, zrow, 0)

        def zslice(sl, carry):
            @pl.when(lax.rem(sl, NS) == sid)
            def _():
                pltpu.sync_copy(zbuf, acc.at[pl.ds(sl * ZROWS, ZROWS)])
            return carry
        lax.fori_loop(0, N // ZROWS, zslice, 0)
        plsc.subcore_barrier()

        pltpu.sync_copy(src_hbm.at[wid], sidx)
        pltpu.sync_copy(dst_hbm.at[wid], didx)

        base_w = wid * EPW

        def issue(c, p, sem):
            cps = (
                pltpu.async_copy(h2t_hbm.at[sidx.at[c]], srows.at[p], sem),
                pltpu.async_copy(h2t_hbm.at[didx.at[c]], drows.at[p], sem),
                pltpu.async_copy(
                    evs_hbm.at[pl.ds(base_w + c * CH, CH)], evc.at[p], sem),
            )
            return cps

        def drain(cps):
            for cp in cps:
                cp.wait()

        def compute(p_static, c, ssem):
            @pl.when(c >= 2)
            def _():
                pltpu.make_async_copy(vals.at[p_static],
                                      acc.at[didx.at[0]], ssem).wait()

            def edge_body(j, ecarry):
                a = (srows[p_static, j, pl.ds(0, H)] *
                     drows[p_static, j, pl.ds(0, H)])
                for dd in range(1, D):
                    a = a + (srows[p_static, j, pl.ds(dd * H, H)] *
                             drows[p_static, j, pl.ds(dd * H, H)])
                a = a + evc[p_static, j, :]
                a = jnp.maximum(a, a * 0.2)
                vals[p_static, j, :] = jnp.exp(a)
                return ecarry
            lax.fori_loop(0, CH, edge_body, 0)
            pltpu.async_copy(vals.at[p_static], acc.at[didx.at[c]], ssem,
                             add=True)

        drain(issue(0, 0, sem0))

        def pair_body(i, carry):
            c0 = i * 2

            @pl.when(c0 + 1 < NCH)
            def _():
                drain_next = issue(c0 + 1, 1, sem1)
            compute(0, c0, sem2)

            @pl.when(c0 + 1 < NCH)
            def _():
                pltpu.make_async_copy(h2t_hbm.at[sidx.at[0]],
                                      srows.at[1], sem1).wait()
                pltpu.make_async_copy(h2t_hbm.at[didx.at[0]],
                                      drows.at[1], sem1).wait()
                pltpu.make_async_copy(
                    evs_hbm.at[pl.ds(base_w, CH)], evc.at[1], sem1).wait()

                @pl.when(c0 + 2 < NCH)
                def _():
                    drain_nn = issue(c0 + 2, 0, sem0)
                compute(1, c0 + 1, sem3)

                @pl.when(c0 + 2 < NCH)
                def _():
                    pltpu.make_async_copy(h2t_hbm.at[sidx.at[0]],
                                          srows.at[0], sem0).wait()
                    pltpu.make_async_copy(h2t_hbm.at[didx.at[0]],
                                          drows.at[0], sem0).wait()
                    pltpu.make_async_copy(
                        evs_hbm.at[pl.ds(base_w, CH)], evc.at[0], sem0).wait()
            return carry
        lax.fori_loop(0, (NCH + 1) // 2, pair_body, 0)

        pltpu.make_async_copy(vals.at[0], acc.at[didx.at[0]], sem2).wait()
        pltpu.make_async_copy(vals.at[1], acc.at[didx.at[0]], sem3).wait()

        plsc.subcore_barrier()

        @pl.when(sid == 0)
        def _():
            pltpu.sync_copy(acc, out_hbm.at[cid])

    return k(h2t, evs, src3, dst3)


def _finish_tc(h2t, partials, w_out_perm):
    NF = N * H // 128

    fold = jnp.tile(jnp.eye(H, dtype=jnp.float32), (8, 8))

    def reduce_body(p_ref, f_ref, s_ref, z_ref):
        s = p_ref[0] + p_ref[1]
        z128 = jnp.sum(s, axis=0, keepdims=True)
        s_ref[...] = s
        z_ref[...] = jnp.dot(z128, f_ref[...],
                             preferred_element_type=jnp.float32,
                             precision=jax.lax.Precision.HIGHEST)

    s_flat, z_t = pl.pallas_call(
        reduce_body,
        in_specs=[pl.BlockSpec(memory_space=pltpu.VMEM),
                  pl.BlockSpec(memory_space=pltpu.VMEM)],
        out_specs=[pl.BlockSpec(memory_space=pltpu.VMEM),
                   pl.BlockSpec(memory_space=pltpu.VMEM)],
        out_shape=[jax.ShapeDtypeStruct((NF, 128), jnp.float32),
                   jax.ShapeDtypeStruct((1, 128), jnp.float32)],
    )(partials.reshape(2, NF, 128), fold)

    BM = 1000
    expand = jnp.tile(jnp.eye(H, dtype=jnp.float32), (1, D))

    def body(h_ref, s_ref, z_ref, t_ref, w_ref, o_ref):
        sn = s_ref[...] / z_ref[...]
        sm = jnp.dot(sn, t_ref[...], preferred_element_type=jnp.float32,
                     precision=jax.lax.Precision.HIGHEST)
        m = h_ref[...] * sm
        o_ref[...] = jnp.maximum(
            jnp.dot(m, w_ref[...], preferred_element_type=jnp.float32,
                    precision=jax.lax.Precision.HIGHEST), 0.0)

    return pl.pallas_call(
        body,
        grid=(N // BM,),
        in_specs=[pl.BlockSpec((BM, HD), lambda i: (i, 0)),
                  pl.BlockSpec((BM, H), lambda i: (i, 0)),
                  pl.BlockSpec((1, H), lambda i: (0, 0)),
                  pl.BlockSpec((H, HD), lambda i: (0, 0)),
                  pl.BlockSpec((HD, D), lambda i: (0, 0))],
        out_specs=pl.BlockSpec((BM, D), lambda i: (i, 0)),
        out_shape=jax.ShapeDtypeStruct((N, D), jnp.float32),
    )(h2t, s_flat.reshape(N, H), z_t[:, :H], expand, w_out_perm)


def kernel(x, edge_attr, edge_index, W, W_edge, W_edge_att, W_att, W_out):
    w_att_perm = jnp.einsum('dj,hk->hdjk', W_att,
                            jnp.eye(H, dtype=jnp.float32)).reshape(HD, HD)
    w_e2_16 = W_edge_att * 16.0
    w_out_perm = W_out.reshape(H, D, D).transpose(1, 0, 2).reshape(HD, D)

    NW = 32
    CH = 80
    src3 = edge_index[0].astype(jnp.int32).reshape(NW, E // (NW * CH), CH)
    dst3 = edge_index[1].astype(jnp.int32).reshape(NW, E // (NW * CH), CH)

    h2t = _matmul2_tc(x, W, w_att_perm, 1000)
    evs = _matmul2_tc(edge_attr, W_edge, w_e2_16, 4000)

    partials = _sc_edge_pass(h2t, evs, src3, dst3)

    return _finish_tc(h2t, partials, w_out_perm)


# speedup vs baseline: 1.2883x; 1.2883x over previous
import functools

import jax
import jax.numpy as jnp
from jax import lax
from jax.experimental import pallas as pl
from jax.experimental.pallas import tpu as pltpu
from jax.experimental.pallas import tpu_sc as plsc

N = 10000
E = 320000
IN_DIM = 128
H = 16
D = 16
HD = H * D


def _matmul2_tc(a, b1, b2, bm):
    M, K = a.shape
    _, K1 = b1.shape
    _, Nc = b2.shape

    def body(a_ref, b1_ref, b2_ref, o_ref):
        t = jnp.dot(a_ref[...].astype(jnp.bfloat16),
                    b1_ref[...].astype(jnp.bfloat16),
                    preferred_element_type=jnp.float32)
        o_ref[...] = jnp.dot(t.astype(jnp.bfloat16),
                             b2_ref[...].astype(jnp.bfloat16),
                             preferred_element_type=jnp.float32)

    return pl.pallas_call(
        body,
        grid=(M // bm,),
        in_specs=[pl.BlockSpec((bm, K), lambda i: (i, 0)),
                  pl.BlockSpec((K, K1), lambda i: (0, 0)),
                  pl.BlockSpec((K1, Nc), lambda i: (0, 0))],
        out_specs=pl.BlockSpec((bm, Nc), lambda i: (i, 0)),
        out_shape=jax.ShapeDtypeStruct((M, Nc), jnp.float32),
    )(a, b1, b2)


def _input_mms_tc(x, w1, w2, ea8, w3, w4):
    BM = 2000
    M1 = x.shape[0] // BM
    M2 = ea8.shape[0] // BM

    def mm2(a, b1_ref, b2_ref):
        t = jnp.dot(a.astype(jnp.bfloat16), b1_ref[...].astype(jnp.bfloat16),
                    preferred_element_type=jnp.float32)
        return jnp.dot(t.astype(jnp.bfloat16),
                       b2_ref[...].astype(jnp.bfloat16),
                       preferred_element_type=jnp.float32)

    def body(x_ref, w1_ref, w2_ref, ea_ref, w3_ref, w4_ref, o1_ref, o2_ref):
        i = pl.program_id(0)

        @pl.when(i < M1)
        def _():
            o1_ref[...] = mm2(x_ref[...], w1_ref, w2_ref)

        @pl.when(i >= M1)
        def _():
            o2_ref[...] = mm2(ea_ref[...], w3_ref, w4_ref)

    return pl.pallas_call(
        body,
        grid=(M1 + M2,),
        in_specs=[
            pl.BlockSpec((BM, 128), lambda i: (jnp.minimum(i, M1 - 1), 0)),
            pl.BlockSpec((128, HD), lambda i: (0, 0)),
            pl.BlockSpec((HD, HD), lambda i: (0, 0)),
            pl.BlockSpec((BM, 128), lambda i: (jnp.maximum(i - M1, 0), 0)),
            pl.BlockSpec((128, 512), lambda i: (0, 0)),
            pl.BlockSpec((512, 128), lambda i: (0, 0)),
        ],
        out_specs=[
            pl.BlockSpec((BM, HD), lambda i: (jnp.minimum(i, M1 - 1), 0)),
            pl.BlockSpec((BM, 128), lambda i: (jnp.maximum(i - M1, 0), 0)),
        ],
        out_shape=[jax.ShapeDtypeStruct((x.shape[0], HD), jnp.float32),
                   jax.ShapeDtypeStruct((ea8.shape[0], 128), jnp.float32)],
    )(x, w1, w2, ea8, w3, w4)


def _sc_edge_pass(h2t, evs, src3, dst3):
    info = plsc.get_sparse_core_info()
    NC, NS = info.num_cores, info.num_subcores
    NW = NC * NS
    EPW = E // NW
    CH = 80
    NCH = EPW // CH
    ZROWS = 250

    mesh = plsc.VectorSubcoreMesh(core_axis_name="c", subcore_axis_name="s")

    @functools.partial(
        pl.kernel, mesh=mesh,
        out_type=jax.ShapeDtypeStruct((NC, N, H), jnp.float32),
        compiler_params=pltpu.CompilerParams(use_tc_tiling_on_sc=False),
        scratch_types=[
            pltpu.VMEM((NCH, CH), jnp.int32),
            pltpu.VMEM((NCH, CH), jnp.int32),
            pltpu.VMEM((2, CH, HD), jnp.float32),
            pltpu.VMEM((2, CH, HD), jnp.float32),
            pltpu.VMEM((2, CH, H), jnp.float32),
            pltpu.VMEM((2, CH, H), jnp.float32),
            pltpu.VMEM((ZROWS, H), jnp.float32),
            pltpu.VMEM_SHARED((N, H), jnp.float32),
            pltpu.SemaphoreType.DMA,
            pltpu.SemaphoreType.DMA,
            pltpu.SemaphoreType.DMA,
            pltpu.SemaphoreType.DMA,
        ],
    )
    def k(h2t_hbm, evs_hbm, src_hbm, dst_hbm, out_hbm,
          sidx, didx, srows, drows, evc, vals, zbuf, acc,
          sem0, sem1, sem2, sem3):
        cid = lax.axis_index("c")
        sid = lax.axis_index("s")
        wid = sid * NC + cid

        def zrow(i, carry):
            zbuf[i, :] = jnp.zeros((H,), jnp.float32)
            return carry
        lax.fori_loop(0, ZROWS, zrow, 0)

        def zslice(sl, carry):
            @pl.when(lax.rem(sl, NS) == sid)
            def _():
                pltpu.sync_copy(zbuf, acc.at[pl.ds(sl * ZROWS, ZROWS)])
            return carry
        lax.fori_loop(0, N // ZROWS, zslice, 0)
        plsc.subcore_barrier()

        pltpu.sync_copy(src_hbm.at[wid], sidx)
        pltpu.sync_copy(dst_hbm.at[wid], didx)

        base_w = wid * EPW

        def issue(c, p, sem):
            cps = (
                pltpu.async_copy(h2t_hbm.at[sidx.at[c]], srows.at[p], sem),
                pltpu.async_copy(h2t_hbm.at[didx.at[c]], drows.at[p], sem),
                pltpu.async_copy(
                    evs_hbm.at[pl.ds(base_w + c * CH, CH)], evc.at[p], sem),
            )
            return cps

        def drain(cps):
            for cp in cps:
                cp.wait()

        def compute(p_static, c, ssem):
            @pl.when(c >= 2)
            def _():
                pltpu.make_async_copy(vals.at[p_static],
                                      acc.at[didx.at[0]], ssem).wait()

            def edge_body(j, ecarry):
                a = (srows[p_static, j, pl.ds(0, H)] *
                     drows[p_static, j, pl.ds(0, H)])
                for dd in range(1, D):
                    a = a + (srows[p_static, j, pl.ds(dd * H, H)] *
                             drows[p_static, j, pl.ds(dd * H, H)])
                a = a + evc[p_static, j, :]
                a = jnp.maximum(a, a * 0.2)
                vals[p_static, j, :] = jnp.exp(a)
                return ecarry
            lax.fori_loop(0, CH, edge_body, 0)
            pltpu.async_copy(vals.at[p_static], acc.at[didx.at[c]], ssem,
                             add=True)

        drain(issue(0, 0, sem0))

        def pair_body(i, carry):
            c0 = i * 2

            @pl.when(c0 + 1 < NCH)
            def _():
                drain_next = issue(c0 + 1, 1, sem1)
            compute(0, c0, sem2)

            @pl.when(c0 + 1 < NCH)
            def _():
                pltpu.make_async_copy(h2t_hbm.at[sidx.at[0]],
                                      srows.at[1], sem1).wait()
                pltpu.make_async_copy(h2t_hbm.at[didx.at[0]],
                                      drows.at[1], sem1).wait()
                pltpu.make_async_copy(
                    evs_hbm.at[pl.ds(base_w, CH)], evc.at[1], sem1).wait()

                @pl.when(c0 + 2 < NCH)
                def _():
                    drain_nn = issue(c0 + 2, 0, sem0)
                compute(1, c0 + 1, sem3)

                @pl.when(c0 + 2 < NCH)
                def _():
                    pltpu.make_async_copy(h2t_hbm.at[sidx.at[0]],
                                          srows.at[0], sem0).wait()
                    pltpu.make_async_copy(h2t_hbm.at[didx.at[0]],
                                          drows.at[0], sem0).wait()
                    pltpu.make_async_copy(
                        evs_hbm.at[pl.ds(base_w, CH)], evc.at[0], sem0).wait()
            return carry
        lax.fori_loop(0, (NCH + 1) // 2, pair_body, 0)

        pltpu.make_async_copy(vals.at[0], acc.at[didx.at[0]], sem2).wait()
        pltpu.make_async_copy(vals.at[1], acc.at[didx.at[0]], sem3).wait()

        plsc.subcore_barrier()

        @pl.when(sid == 0)
        def _():
            pltpu.sync_copy(acc, out_hbm.at[cid])

    return k(h2t, evs, src3, dst3)


def _finish_tc(h2t, partials, w_out_perm):
    NF = N * H // 128

    fold = jnp.tile(jnp.eye(H, dtype=jnp.float32), (8, 8))

    def reduce_body(p_ref, f_ref, s_ref, z_ref):
        s = p_ref[0] + p_ref[1]
        z128 = jnp.sum(s, axis=0, keepdims=True)
        s_ref[...] = s
        z_ref[...] = jnp.dot(z128, f_ref[...],
                             preferred_element_type=jnp.float32,
                             precision=jax.lax.Precision.HIGHEST)

    s_flat, z_t = pl.pallas_call(
        reduce_body,
        in_specs=[pl.BlockSpec(memory_space=pltpu.VMEM),
                  pl.BlockSpec(memory_space=pltpu.VMEM)],
        out_specs=[pl.BlockSpec(memory_space=pltpu.VMEM),
                   pl.BlockSpec(memory_space=pltpu.VMEM)],
        out_shape=[jax.ShapeDtypeStruct((NF, 128), jnp.float32),
                   jax.ShapeDtypeStruct((1, 128), jnp.float32)],
    )(partials.reshape(2, NF, 128), fold)

    BM = 1000
    expand = jnp.tile(jnp.eye(H, dtype=jnp.float32), (1, D))

    def body(h_ref, s_ref, z_ref, t_ref, w_ref, o_ref):
        sn = s_ref[...] / z_ref[...]
        sm = jnp.dot(sn, t_ref[...], preferred_element_type=jnp.float32,
                     precision=jax.lax.Precision.HIGHEST)
        m = h_ref[...] * sm
        o_ref[...] = jnp.maximum(
            jnp.dot(m, w_ref[...], preferred_element_type=jnp.float32,
                    precision=jax.lax.Precision.HIGHEST), 0.0)

    return pl.pallas_call(
        body,
        grid=(N // BM,),
        in_specs=[pl.BlockSpec((BM, HD), lambda i: (i, 0)),
                  pl.BlockSpec((BM, H), lambda i: (i, 0)),
                  pl.BlockSpec((1, H), lambda i: (0, 0)),
                  pl.BlockSpec((H, HD), lambda i: (0, 0)),
                  pl.BlockSpec((HD, D), lambda i: (0, 0))],
        out_specs=pl.BlockSpec((BM, D), lambda i: (i, 0)),
        out_shape=jax.ShapeDtypeStruct((N, D), jnp.float32),
    )(h2t, s_flat.reshape(N, H), z_t[:, :H], expand, w_out_perm)


def kernel(x, edge_attr, edge_index, W, W_edge, W_edge_att, W_att, W_out):
    w_att_perm = jnp.einsum('dj,hk->hdjk', W_att,
                            jnp.eye(H, dtype=jnp.float32)).reshape(HD, HD)
    w_e1 = jnp.kron(jnp.eye(8, dtype=jnp.float32), W_edge)
    w_e2 = jnp.kron(jnp.eye(8, dtype=jnp.float32), W_edge_att * 16.0)
    w_out_perm = W_out.reshape(H, D, D).transpose(1, 0, 2).reshape(HD, D)

    NW = 32
    CH = 80
    src3 = edge_index[0].astype(jnp.int32).reshape(NW, E // (NW * CH), CH)
    dst3 = edge_index[1].astype(jnp.int32).reshape(NW, E // (NW * CH), CH)

    h2t, evs8 = _input_mms_tc(x, W, w_att_perm,
                              edge_attr.reshape(E // 8, 128), w_e1, w_e2)
    evs = evs8.reshape(E, H)

    partials = _sc_edge_pass(h2t, evs, src3, dst3)

    return _finish_tc(h2t, partials, w_out_perm)


# speedup vs baseline: 1.3047x; 1.0128x over previous
import functools

import jax
import jax.numpy as jnp
from jax import lax
from jax.experimental import pallas as pl
from jax.experimental.pallas import tpu as pltpu
from jax.experimental.pallas import tpu_sc as plsc

N = 10000
E = 320000
IN_DIM = 128
H = 16
D = 16
HD = H * D


def _matmul2_tc(a, b1, b2, bm):
    M, K = a.shape
    _, K1 = b1.shape
    _, Nc = b2.shape

    def body(a_ref, b1_ref, b2_ref, o_ref):
        t = jnp.dot(a_ref[...].astype(jnp.bfloat16),
                    b1_ref[...].astype(jnp.bfloat16),
                    preferred_element_type=jnp.float32)
        o_ref[...] = jnp.dot(t.astype(jnp.bfloat16),
                             b2_ref[...].astype(jnp.bfloat16),
                             preferred_element_type=jnp.float32)

    return pl.pallas_call(
        body,
        grid=(M // bm,),
        in_specs=[pl.BlockSpec((bm, K), lambda i: (i, 0)),
                  pl.BlockSpec((K, K1), lambda i: (0, 0)),
                  pl.BlockSpec((K1, Nc), lambda i: (0, 0))],
        out_specs=pl.BlockSpec((bm, Nc), lambda i: (i, 0)),
        out_shape=jax.ShapeDtypeStruct((M, Nc), jnp.float32),
    )(a, b1, b2)


def _sc_edge_pass(h2t, evs, src3, dst3):
    info = plsc.get_sparse_core_info()
    NC, NS = info.num_cores, info.num_subcores
    NW = NC * NS
    EPW = E // NW
    CH = 80
    NCH = EPW // CH
    ZROWS = 250

    mesh = plsc.VectorSubcoreMesh(core_axis_name="c", subcore_axis_name="s")

    @functools.partial(
        pl.kernel, mesh=mesh,
        out_type=jax.ShapeDtypeStruct((NC, N, H), jnp.float32),
        compiler_params=pltpu.CompilerParams(use_tc_tiling_on_sc=False),
        scratch_types=[
            pltpu.VMEM((NCH, CH), jnp.int32),
            pltpu.VMEM((NCH, CH), jnp.int32),
            pltpu.VMEM((2, CH, HD), jnp.float32),
            pltpu.VMEM((2, CH, HD), jnp.float32),
            pltpu.VMEM((2, CH, H), jnp.float32),
            pltpu.VMEM((2, CH, H), jnp.float32),
            pltpu.VMEM((ZROWS, H), jnp.float32),
            pltpu.VMEM_SHARED((N, H), jnp.float32),
            pltpu.SemaphoreType.DMA,
            pltpu.SemaphoreType.DMA,
            pltpu.SemaphoreType.DMA,
            pltpu.SemaphoreType.DMA,
        ],
    )
    def k(h2t_hbm, evs_hbm, src_hbm, dst_hbm, out_hbm,
          sidx, didx, srows, drows, evc, vals, zbuf, acc,
          sem0, sem1, sem2, sem3):
        cid = lax.axis_index("c")
        sid = lax.axis_index("s")
        wid = sid * NC + cid

        def zrow(i, carry):
            zbuf[i, :] = jnp.zeros((H,), jnp.float32)
            return carry
        lax.fori_loop(0, ZROWS, zrow, 0)

        def zslice(sl, carry):
            @pl.when(lax.rem(sl, NS) == sid)
            def _():
                pltpu.sync_copy(zbuf, acc.at[pl.ds(sl * ZROWS, ZROWS)])
            return carry
        lax.fori_loop(0, N // ZROWS, zslice, 0)
        plsc.subcore_barrier()

        pltpu.sync_copy(src_hbm.at[wid], sidx)
        pltpu.sync_copy(dst_hbm.at[wid], didx)

        base_w = wid * EPW

        def issue(c, p, sem):
            cps = (
                pltpu.async_copy(h2t_hbm.at[sidx.at[c]], srows.at[p], sem),
                pltpu.async_copy(h2t_hbm.at[didx.at[c]], drows.at[p], sem),
                pltpu.async_copy(
                    evs_hbm.at[pl.ds(base_w + c * CH, CH)], evc.at[p], sem),
            )
            return cps

        def drain(cps):
            for cp in cps:
                cp.wait()

        def compute(p_static, c, ssem):
            @pl.when(c >= 2)
            def _():
                pltpu.make_async_copy(vals.at[p_static],
                                      acc.at[didx.at[0]], ssem).wait()

            def edge_body(j, ecarry):
                a = (srows[p_static, j, pl.ds(0, H)] *
                     drows[p_static, j, pl.ds(0, H)])
                for dd in range(1, D):
                    a = a + (srows[p_static, j, pl.ds(dd * H, H)] *
                             drows[p_static, j, pl.ds(dd * H, H)])
                a = a + evc[p_static, j, :]
                a = jnp.maximum(a, a * 0.2)
                vals[p_static, j, :] = jnp.exp(a)
                return ecarry
            lax.fori_loop(0, CH, edge_body, 0)
            pltpu.async_copy(vals.at[p_static], acc.at[didx.at[c]], ssem,
                             add=True)

        drain(issue(0, 0, sem0))

        def pair_body(i, carry):
            c0 = i * 2

            @pl.when(c0 + 1 < NCH)
            def _():
                drain_next = issue(c0 + 1, 1, sem1)
            compute(0, c0, sem2)

            @pl.when(c0 + 1 < NCH)
            def _():
                pltpu.make_async_copy(h2t_hbm.at[sidx.at[0]],
                                      srows.at[1], sem1).wait()
                pltpu.make_async_copy(h2t_hbm.at[didx.at[0]],
                                      drows.at[1], sem1).wait()
                pltpu.make_async_copy(
                    evs_hbm.at[pl.ds(base_w, CH)], evc.at[1], sem1).wait()

                @pl.when(c0 + 2 < NCH)
                def _():
                    drain_nn = issue(c0 + 2, 0, sem0)
                compute(1, c0 + 1, sem3)

                @pl.when(c0 + 2 < NCH)
                def _():
                    pltpu.make_async_copy(h2t_hbm.at[sidx.at[0]],
                                          srows.at[0], sem0).wait()
                    pltpu.make_async_copy(h2t_hbm.at[didx.at[0]],
                                          drows.at[0], sem0).wait()
                    pltpu.make_async_copy(
                        evs_hbm.at[pl.ds(base_w, CH)], evc.at[0], sem0).wait()
            return carry
        lax.fori_loop(0, (NCH + 1) // 2, pair_body, 0)

        pltpu.make_async_copy(vals.at[0], acc.at[didx.at[0]], sem2).wait()
        pltpu.make_async_copy(vals.at[1], acc.at[didx.at[0]], sem3).wait()

        plsc.subcore_barrier()

        @pl.when(sid == 0)
        def _():
            pltpu.sync_copy(acc, out_hbm.at[cid])

    return k(h2t, evs, src3, dst3)


def _finish_tc(h2t, partials, w_out_perm):
    NF = N * H // 128

    fold = jnp.tile(jnp.eye(H, dtype=jnp.float32), (8, 8))

    def reduce_body(p_ref, f_ref, s_ref, z_ref):
        s = p_ref[0] + p_ref[1]
        z128 = jnp.sum(s, axis=0, keepdims=True)
        s_ref[...] = s
        z_ref[...] = jnp.dot(z128, f_ref[...],
                             preferred_element_type=jnp.float32,
                             precision=jax.lax.Precision.HIGHEST)

    s_flat, z_t = pl.pallas_call(
        reduce_body,
        in_specs=[pl.BlockSpec(memory_space=pltpu.VMEM),
                  pl.BlockSpec(memory_space=pltpu.VMEM)],
        out_specs=[pl.BlockSpec(memory_space=pltpu.VMEM),
                   pl.BlockSpec(memory_space=pltpu.VMEM)],
        out_shape=[jax.ShapeDtypeStruct((NF, 128), jnp.float32),
                   jax.ShapeDtypeStruct((1, 128), jnp.float32)],
    )(partials.reshape(2, NF, 128), fold)

    BM = 1000
    expand = jnp.tile(jnp.eye(H, dtype=jnp.float32), (1, D))

    def body(h_ref, s_ref, z_ref, t_ref, w_ref, o_ref):
        sn = s_ref[...] / z_ref[...]
        sm = jnp.dot(sn, t_ref[...], preferred_element_type=jnp.float32,
                     precision=jax.lax.Precision.HIGHEST)
        m = h_ref[...] * sm
        o_ref[...] = jnp.maximum(
            jnp.dot(m, w_ref[...], preferred_element_type=jnp.float32,
                    precision=jax.lax.Precision.HIGHEST), 0.0)

    return pl.pallas_call(
        body,
        grid=(N // BM,),
        in_specs=[pl.BlockSpec((BM, HD), lambda i: (i, 0)),
                  pl.BlockSpec((BM, H), lambda i: (i, 0)),
                  pl.BlockSpec((1, H), lambda i: (0, 0)),
                  pl.BlockSpec((H, HD), lambda i: (0, 0)),
                  pl.BlockSpec((HD, D), lambda i: (0, 0))],
        out_specs=pl.BlockSpec((BM, D), lambda i: (i, 0)),
        out_shape=jax.ShapeDtypeStruct((N, D), jnp.float32),
    )(h2t, s_flat.reshape(N, H), z_t[:, :H], expand, w_out_perm)


def kernel(x, edge_attr, edge_index, W, W_edge, W_edge_att, W_att, W_out):
    w_att_perm = jnp.einsum('dj,hk->hdjk', W_att,
                            jnp.eye(H, dtype=jnp.float32)).reshape(HD, HD)
    w_e1 = jnp.kron(jnp.eye(8, dtype=jnp.float32), W_edge)
    w_e2 = jnp.kron(jnp.eye(8, dtype=jnp.float32), W_edge_att * 16.0)
    w_out_perm = W_out.reshape(H, D, D).transpose(1, 0, 2).reshape(HD, D)

    NW = 32
    CH = 80
    src3 = edge_index[0].astype(jnp.int32).reshape(NW, E // (NW * CH), CH)
    dst3 = edge_index[1].astype(jnp.int32).reshape(NW, E // (NW * CH), CH)

    h2t = _matmul2_tc(x, W, w_att_perm, 1000)
    evs = _matmul2_tc(edge_attr.reshape(E // 8, 128), w_e1, w_e2,
                      2000).reshape(E, H)

    partials = _sc_edge_pass(h2t, evs, src3, dst3)

    return _finish_tc(h2t, partials, w_out_perm)
